# Initial kernel scaffold; baseline (speedup 1.0000x reference)
import jax
import jax.numpy as jnp
from jax.experimental import pallas as pl


def kernel(x, table):
    raise NotImplementedError("write your pallas kernel here")


# speedup vs baseline: 4.8310x; 4.8310x over previous
import functools

import jax
import jax.numpy as jnp
from jax import lax
from jax.experimental import pallas as pl
from jax.experimental.pallas import tpu as pltpu
from jax.experimental.pallas import tpu_sc as plsc

_NC = 2
_NS = 16
_NW = _NC * _NS


def _emb_call(total, D, C):
    n_chunks = total // (_NW * C)
    b_per_w = total // _NW
    mesh = plsc.VectorSubcoreMesh(core_axis_name="c", subcore_axis_name="s",
                                  num_cores=_NC, num_subcores=_NS)

    @functools.partial(
        pl.kernel,
        out_type=jax.ShapeDtypeStruct((total, D), jnp.float32),
        mesh=mesh,
        scratch_types=[
            pltpu.VMEM((C,), jnp.int32),
            pltpu.VMEM((C, D), jnp.float32),
            pltpu.SemaphoreType.DMA,
        ],
        compiler_params=pltpu.CompilerParams(use_tc_tiling_on_sc=False),
    )
    def emb(x_hbm, table_hbm, out_hbm, idx_v, rows_v, sem):
        wid = lax.axis_index("s") * _NC + lax.axis_index("c")
        base = wid * b_per_w

        def body(g, carry):
            off = base + g * C
            pltpu.sync_copy(x_hbm.at[pl.ds(off, C)], idx_v)
            pltpu.async_copy(table_hbm.at[idx_v], rows_v, sem).wait()
            pltpu.sync_copy(rows_v, out_hbm.at[pl.ds(off, C)])
            return carry

        lax.fori_loop(0, n_chunks, body, 0)

    return emb


def kernel(x, table):
    B, H = x.shape
    V, D = table.shape
    total = B * H
    C = 1024
    out = _emb_call(total, D, C)(x.reshape(total), table)
    return out.reshape(B, H, D)


# speedup vs baseline: 5.0811x; 1.0518x over previous
import functools

import jax
import jax.numpy as jnp
from jax import lax
from jax.experimental import pallas as pl
from jax.experimental.pallas import tpu as pltpu
from jax.experimental.pallas import tpu_sc as plsc

_NC = 2
_NS = 16
_NW = _NC * _NS
_C = 1600


def _emb_call(total, D):
    C = _C
    b_per_w = total // _NW
    n_chunks = b_per_w // C
    assert b_per_w % C == 0 and n_chunks >= 8 and n_chunks % 4 == 0
    mesh = plsc.VectorSubcoreMesh(core_axis_name="c", subcore_axis_name="s",
                                  num_cores=_NC, num_subcores=_NS)

    @functools.partial(
        pl.kernel,
        out_type=jax.ShapeDtypeStruct((total, D), jnp.float32),
        mesh=mesh,
        scratch_types=[
            pltpu.VMEM((4, C), jnp.int32),
            pltpu.VMEM((2, C, D), jnp.float32),
            pltpu.SemaphoreType.DMA,
            pltpu.SemaphoreType.DMA,
            pltpu.SemaphoreType.DMA,
            pltpu.SemaphoreType.DMA,
            pltpu.SemaphoreType.DMA,
            pltpu.SemaphoreType.DMA,
            pltpu.SemaphoreType.DMA,
            pltpu.SemaphoreType.DMA,
        ],
        compiler_params=pltpu.CompilerParams(use_tc_tiling_on_sc=False),
    )
    def emb(x_hbm, table_hbm, out_hbm, idx_v, rows_v,
            is0, is1, is2, is3, gs0, gs1, os0, os1):
        isems = (is0, is1, is2, is3)
        gsems = (gs0, gs1)
        osems = (os0, os1)
        wid = lax.axis_index("s") * _NC + lax.axis_index("c")
        base = wid * b_per_w

        def ds(g):
            return pl.ds(base + g * C, C)

        def stage_idx(g, slot):
            pltpu.async_copy(x_hbm.at[ds(g)], idx_v.at[slot], isems[slot])

        def wait_idx(slot):
            pltpu.make_async_copy(x_hbm.at[pl.ds(base, C)],
                                  idx_v.at[slot], isems[slot]).wait()

        def fire_gather(b, slot):
            pltpu.async_copy(table_hbm.at[idx_v.at[slot]], rows_v.at[b],
                             gsems[b])

        def wait_gather(b):
            pltpu.make_async_copy(out_hbm.at[pl.ds(base, C)],
                                  rows_v.at[b], gsems[b]).wait()

        def fire_out(g, b):
            pltpu.async_copy(rows_v.at[b], out_hbm.at[ds(g)], osems[b])

        def wait_out(b):
            pltpu.make_async_copy(rows_v.at[b],
                                  out_hbm.at[pl.ds(base, C)], osems[b]).wait()

        def step(g, j, *, w_out, w_prev, do_stage):
            b = j % 2
            wait_idx(j)
            if w_out:
                wait_out(b)
            fire_gather(b, j)
            if w_prev:
                wait_gather(1 - b)
                fire_out(g - 1, 1 - b)
            if do_stage:
                stage_idx(g + 2, (j + 2) % 4)

        stage_idx(0, 0)
        stage_idx(1, 1)
        step(0, 0, w_out=False, w_prev=False, do_stage=True)
        step(1, 1, w_out=False, w_prev=True, do_stage=True)
        step(2, 2, w_out=True, w_prev=True, do_stage=True)
        step(3, 3, w_out=True, w_prev=True, do_stage=True)

        @pl.loop(1, n_chunks // 4 - 1)
        def _(o):
            g0 = o * 4
            for j in range(4):
                step(g0 + j, j, w_out=True, w_prev=True, do_stage=True)

        gl = n_chunks - 4
        step(gl + 0, 0, w_out=True, w_prev=True, do_stage=True)
        step(gl + 1, 1, w_out=True, w_prev=True, do_stage=True)
        step(gl + 2, 2, w_out=True, w_prev=True, do_stage=False)
        step(gl + 3, 3, w_out=True, w_prev=True, do_stage=False)
        wait_gather(1)
        fire_out(n_chunks - 1, 1)
        wait_out(0)
        wait_out(1)

    return emb


def kernel(x, table):
    B, H = x.shape
    V, D = table.shape
    total = B * H
    out = _emb_call(total, D)(x.reshape(total), table)
    return out.reshape(B, H, D)
